# Initial kernel scaffold; baseline (speedup 1.0000x reference)
import jax
import jax.numpy as jnp
from jax.experimental import pallas as pl


def kernel(x):
    raise NotImplementedError("write your pallas kernel here")


# speedup vs baseline: 5.8205x; 5.8205x over previous
import numpy as np
import jax
import jax.numpy as jnp
from jax.experimental import pallas as pl
from jax.experimental.pallas import tpu as pltpu

_NUM_SAMPLES = (16, 32)
_TEMPORAL_DIM = 1


def _subsample_indices(T, t):
    w = np.arange(t - 1, dtype=np.float32) / np.float32(t - 1)
    body = np.float32(0.0) * (np.float32(1.0) - w) + np.float32(T - 1) * w
    vals = np.concatenate([body, np.asarray([T - 1], np.float32)])
    vals = np.clip(vals, 0, T - 1)
    return vals.astype(np.int32)


def _copy_body(idx_ref, x_ref, o_ref):
    o_ref[...] = x_ref[...]


def _take_temporal(x, idx):
    B, T, C, H, W = x.shape
    t = int(idx.shape[0])
    return pl.pallas_call(
        _copy_body,
        grid_spec=pltpu.PrefetchScalarGridSpec(
            num_scalar_prefetch=1,
            grid=(B, t),
            in_specs=[pl.BlockSpec(
                (1, 1, C, H, W),
                lambda b, i, idx_ref: (b, idx_ref[i], 0, 0, 0))],
            out_specs=pl.BlockSpec(
                (1, 1, C, H, W),
                lambda b, i, idx_ref: (b, i, 0, 0, 0)),
        ),
        out_shape=jax.ShapeDtypeStruct((B, t, C, H, W), x.dtype),
    )(jnp.asarray(idx, dtype=jnp.int32), x)


def kernel(x):
    T = x.shape[_TEMPORAL_DIM]
    return tuple(_take_temporal(x, _subsample_indices(T, t))
                 for t in _NUM_SAMPLES)


# speedup vs baseline: 8.7296x; 1.4998x over previous
import functools

import numpy as np
import jax
import jax.numpy as jnp
from jax import lax
from jax.experimental import pallas as pl
from jax.experimental.pallas import tpu as pltpu
from jax.experimental.pallas import tpu_sc as plsc

_NUM_SAMPLES = (16, 32)
_TEMPORAL_DIM = 1

_NC = 2
_NS = 16
_NW = _NC * _NS
_NBUF = 4


def _subsample_indices(T, t):
    w = np.arange(t - 1, dtype=np.float32) / np.float32(t - 1)
    body = np.float32(0.0) * (np.float32(1.0) - w) + np.float32(T - 1) * w
    vals = np.concatenate([body, np.asarray([T - 1], np.float32)])
    vals = np.clip(vals, 0, T - 1)
    return vals.astype(np.int32)


def kernel(x):
    B, T, C, H, W = x.shape
    t16, t32 = _NUM_SAMPLES
    for t in (t16, t32):
        assert all(int(v) == (j * (T - 1)) // (t - 1)
                   for j, v in enumerate(_subsample_indices(T, t)))

    HH = H // 2
    xr = x.reshape(B * T * C * 2, HH, W)
    Q16 = B * t16 * C * 2
    Q32 = B * t32 * C * 2
    pw16 = Q16 // _NW
    pw32 = Q32 // _NW

    mesh = plsc.VectorSubcoreMesh(core_axis_name="c", subcore_axis_name="s")

    @functools.partial(
        pl.kernel,
        mesh=mesh,
        out_type=[jax.ShapeDtypeStruct((Q16, HH, W), x.dtype),
                  jax.ShapeDtypeStruct((Q32, HH, W), x.dtype)],
        scratch_types=(
            [pltpu.VMEM((1, HH, W), x.dtype) for _ in range(_NBUF)]
            + [pltpu.SemaphoreType.DMA for _ in range(2 * _NBUF)]
        ),
    )
    def run(x_hbm, o16_hbm, o32_hbm, *scratch):
        bufs = scratch[:_NBUF]
        gsems = scratch[_NBUF:2 * _NBUF]
        ssems = scratch[2 * _NBUF:3 * _NBUF]
        wid = lax.axis_index("s") * _NC + lax.axis_index("c")

        def phase(o_hbm, nj, n_chunks):
            base = wid * n_chunks

            def src_chunk(q):
                r = q // 2
                h = q % 2
                b = r // (nj * C)
                rem = r % (nj * C)
                j = rem // C
                c = rem % C
                tsrc = ((T - 1) * j) // (nj - 1)
                return ((b * T + tsrc) * C + c) * 2 + h

            def g_copy(q, slot):
                return pltpu.make_async_copy(
                    x_hbm.at[pl.ds(src_chunk(base + q), 1)],
                    bufs[slot], gsems[slot])

            def s_copy(q, slot):
                return pltpu.make_async_copy(
                    bufs[slot], o_hbm.at[pl.ds(base + q, 1)], ssems[slot])

            niter = n_chunks // _NBUF
            for slot in range(_NBUF):
                g_copy(slot, slot).start()

            def body(i, carry):
                q0 = i * _NBUF
                for slot in range(_NBUF):
                    g_copy(q0 + slot, slot).wait()
                    s_copy(q0 + slot, slot).start()
                for slot in range(_NBUF):
                    s_copy(q0 + slot, slot).wait()
                    g_copy(q0 + _NBUF + slot, slot).start()
                return carry

            lax.fori_loop(0, niter - 1, body, 0)
            qL = (niter - 1) * _NBUF
            for slot in range(_NBUF):
                g_copy(qL + slot, slot).wait()
                s_copy(qL + slot, slot).start()
            for slot in range(_NBUF):
                s_copy(qL + slot, slot).wait()

        phase(o16_hbm, t16, pw16)
        phase(o32_hbm, t32, pw32)

    o16, o32 = run(xr)
    return (o16.reshape(B, t16, C, H, W), o32.reshape(B, t32, C, H, W))


# speedup vs baseline: 8.7591x; 1.0034x over previous
import functools

import numpy as np
import jax
import jax.numpy as jnp
from jax import lax
from jax.experimental import pallas as pl
from jax.experimental.pallas import tpu as pltpu
from jax.experimental.pallas import tpu_sc as plsc

_NUM_SAMPLES = (16, 32)
_TEMPORAL_DIM = 1

_NC = 2
_NS = 16
_NW = _NC * _NS
_NBUF = 8
_HSPLIT = 4


def _subsample_indices(T, t):
    w = np.arange(t - 1, dtype=np.float32) / np.float32(t - 1)
    body = np.float32(0.0) * (np.float32(1.0) - w) + np.float32(T - 1) * w
    vals = np.concatenate([body, np.asarray([T - 1], np.float32)])
    vals = np.clip(vals, 0, T - 1)
    return vals.astype(np.int32)


def kernel(x):
    B, T, C, H, W = x.shape
    t16, t32 = _NUM_SAMPLES
    for t in (t16, t32):
        assert all(int(v) == (j * (T - 1)) // (t - 1)
                   for j, v in enumerate(_subsample_indices(T, t)))

    S = _HSPLIT
    HH = H // S
    xr = x.reshape(B * T * C * S, HH, W)
    Q16 = B * t16 * C * S
    Q32 = B * t32 * C * S
    pw16 = Q16 // _NW
    pw32 = Q32 // _NW

    mesh = plsc.VectorSubcoreMesh(core_axis_name="c", subcore_axis_name="s")

    @functools.partial(
        pl.kernel,
        mesh=mesh,
        out_type=[jax.ShapeDtypeStruct((Q16, HH, W), x.dtype),
                  jax.ShapeDtypeStruct((Q32, HH, W), x.dtype)],
        scratch_types=(
            [pltpu.VMEM((1, HH, W), x.dtype) for _ in range(_NBUF)]
            + [pltpu.SemaphoreType.DMA for _ in range(2 * _NBUF)]
        ),
    )
    def run(x_hbm, o16_hbm, o32_hbm, *scratch):
        bufs = scratch[:_NBUF]
        gsems = scratch[_NBUF:2 * _NBUF]
        ssems = scratch[2 * _NBUF:3 * _NBUF]
        wid = lax.axis_index("s") * _NC + lax.axis_index("c")

        def phase(o_hbm, nj, n_chunks):
            base = wid * n_chunks

            def src_chunk(q):
                r = q // S
                h = q % S
                b = r // (nj * C)
                rem = r % (nj * C)
                j = rem // C
                c = rem % C
                tsrc = ((T - 1) * j) // (nj - 1)
                return ((b * T + tsrc) * C + c) * S + h

            def g_copy(q, slot):
                return pltpu.make_async_copy(
                    x_hbm.at[pl.ds(src_chunk(base + q), 1)],
                    bufs[slot], gsems[slot])

            def s_copy(q, slot):
                return pltpu.make_async_copy(
                    bufs[slot], o_hbm.at[pl.ds(base + q, 1)], ssems[slot])

            niter = n_chunks // _NBUF
            for slot in range(_NBUF):
                g_copy(slot, slot).start()

            def body(i, carry):
                q0 = i * _NBUF
                for slot in range(_NBUF):
                    g_copy(q0 + slot, slot).wait()
                    s_copy(q0 + slot, slot).start()
                for slot in range(_NBUF):
                    s_copy(q0 + slot, slot).wait()
                    g_copy(q0 + _NBUF + slot, slot).start()
                return carry

            lax.fori_loop(0, niter - 1, body, 0)
            qL = (niter - 1) * _NBUF
            for slot in range(_NBUF):
                g_copy(qL + slot, slot).wait()
                s_copy(qL + slot, slot).start()
            for slot in range(_NBUF):
                s_copy(qL + slot, slot).wait()

        phase(o16_hbm, t16, pw16)
        phase(o32_hbm, t32, pw32)

    o16, o32 = run(xr)
    return (o16.reshape(B, t16, C, H, W), o32.reshape(B, t32, C, H, W))


# speedup vs baseline: 8.9524x; 1.0221x over previous
import functools

import numpy as np
import jax
import jax.numpy as jnp
from jax import lax
from jax.experimental import pallas as pl
from jax.experimental.pallas import tpu as pltpu
from jax.experimental.pallas import tpu_sc as plsc

_NUM_SAMPLES = (16, 32)
_TEMPORAL_DIM = 1

_NC = 2
_NS = 16
_NW = _NC * _NS
_NBUF = 8
_HSPLIT = 4


def _subsample_indices(T, t):
    w = np.arange(t - 1, dtype=np.float32) / np.float32(t - 1)
    body = np.float32(0.0) * (np.float32(1.0) - w) + np.float32(T - 1) * w
    vals = np.concatenate([body, np.asarray([T - 1], np.float32)])
    vals = np.clip(vals, 0, T - 1)
    return vals.astype(np.int32)


def _tc_copy_body(idx_ref, x_ref, o_ref):
    o_ref[...] = x_ref[...]


def _tc_take_temporal(x, idx):
    B, T, C, H, W = x.shape
    t = int(idx.shape[0])
    return pl.pallas_call(
        _tc_copy_body,
        grid_spec=pltpu.PrefetchScalarGridSpec(
            num_scalar_prefetch=1,
            grid=(B, t),
            in_specs=[pl.BlockSpec(
                (1, 1, C, H, W),
                lambda b, i, idx_ref: (b, idx_ref[i], 0, 0, 0))],
            out_specs=pl.BlockSpec(
                (1, 1, C, H, W),
                lambda b, i, idx_ref: (b, i, 0, 0, 0)),
        ),
        out_shape=jax.ShapeDtypeStruct((B, t, C, H, W), x.dtype),
    )(jnp.asarray(idx, dtype=jnp.int32), x)


def _sc_take_temporal(x, nj):
    B, T, C, H, W = x.shape
    S = _HSPLIT
    HH = H // S
    xr = x.reshape(B * T * C * S, HH, W)
    Q = B * nj * C * S
    pw = Q // _NW

    mesh = plsc.VectorSubcoreMesh(core_axis_name="c", subcore_axis_name="s")

    @functools.partial(
        pl.kernel,
        mesh=mesh,
        out_type=jax.ShapeDtypeStruct((Q, HH, W), x.dtype),
        scratch_types=(
            [pltpu.VMEM((1, HH, W), x.dtype) for _ in range(_NBUF)]
            + [pltpu.SemaphoreType.DMA for _ in range(2 * _NBUF)]
        ),
    )
    def run(x_hbm, o_hbm, *scratch):
        bufs = scratch[:_NBUF]
        gsems = scratch[_NBUF:2 * _NBUF]
        ssems = scratch[2 * _NBUF:3 * _NBUF]
        wid = lax.axis_index("s") * _NC + lax.axis_index("c")
        base = wid * pw

        def src_chunk(q):
            r = q // S
            h = q % S
            b = r // (nj * C)
            rem = r % (nj * C)
            j = rem // C
            c = rem % C
            tsrc = ((T - 1) * j) // (nj - 1)
            return ((b * T + tsrc) * C + c) * S + h

        def g_copy(q, slot):
            return pltpu.make_async_copy(
                x_hbm.at[pl.ds(src_chunk(base + q), 1)],
                bufs[slot], gsems[slot])

        def s_copy(q, slot):
            return pltpu.make_async_copy(
                bufs[slot], o_hbm.at[pl.ds(base + q, 1)], ssems[slot])

        niter = pw // _NBUF
        for slot in range(_NBUF):
            g_copy(slot, slot).start()

        def body(i, carry):
            q0 = i * _NBUF
            for slot in range(_NBUF):
                g_copy(q0 + slot, slot).wait()
                s_copy(q0 + slot, slot).start()
            for slot in range(_NBUF):
                s_copy(q0 + slot, slot).wait()
                g_copy(q0 + _NBUF + slot, slot).start()
            return carry

        lax.fori_loop(0, niter - 1, body, 0)
        qL = (niter - 1) * _NBUF
        for slot in range(_NBUF):
            g_copy(qL + slot, slot).wait()
            s_copy(qL + slot, slot).start()
        for slot in range(_NBUF):
            s_copy(qL + slot, slot).wait()

    return run(xr).reshape(B, nj, C, H, W)


def kernel(x):
    B, T, C, H, W = x.shape
    t16, t32 = _NUM_SAMPLES
    for t in (t16, t32):
        assert all(int(v) == (j * (T - 1)) // (t - 1)
                   for j, v in enumerate(_subsample_indices(T, t)))
    o16 = _tc_take_temporal(x, _subsample_indices(T, t16))
    o32 = _sc_take_temporal(x, t32)
    return (o16, o32)
